# baseline (speedup 1.0000x reference)
import jax
import jax.numpy as jnp
from jax import lax
from jax.experimental import pallas as pl
from jax.experimental.pallas import tpu as pltpu
from jax.experimental.pallas import tpu_sc as plsc

NC = 2
NS = 16
L = 16
NW = NC * NS
ROWS = 64
COLS = 32768

SC_COLS = 4096
RPT = ROWS // NW

UR = 8
U = UR * RPT
NB = 1
BLK = SC_COLS // NB
GPB = BLK // (UR * L)

TC_BLK = 4096
TC_NCB = (COLS - SC_COLS) // TC_BLK


def _lane_shuffle(v, perm):
  dnums = lax.GatherDimensionNumbers(
      offset_dims=(), collapsed_slice_dims=(0,), start_index_map=(0,))
  return lax.gather(v, perm[:, None], dnums, slice_sizes=(1,),
                    mode=lax.GatherScatterMode.PROMISE_IN_BOUNDS)


def _sc_body(x_hbm, vals_hbm, idxs_hbm, xv, val_v, idx_v, *sems):
  cid = lax.axis_index("c")
  sid = lax.axis_index("s")
  row0 = (cid * NS + sid) * RPT

  descs = []
  for r in range(RPT):
    for b in range(NB):
      d = pltpu.async_copy(
          x_hbm.at[pl.ds(row0 + r, 1), pl.ds(b * BLK, BLK)],
          xv.at[pl.ds(r, 1), pl.ds(b * BLK, BLK)],
          sems[r * NB + b])
      descs.append(d)

  iota = lax.iota(jnp.int32, L)

  rms = [jnp.full((L,), -jnp.inf, jnp.float32) for _ in range(U)]
  rcs = [jnp.zeros((L,), jnp.int32) for _ in range(U)]

  for b in range(NB):
    for r in range(RPT):
      descs[r * NB + b].wait()

    def scan_step(g, carry):
      rms = list(carry[:U])
      rcs = list(carry[U:])
      gvec = jnp.zeros((L,), jnp.int32) + g
      base = g * (UR * L)
      for k in range(U):
        r, kk = divmod(k, UR)
        chunk = xv[r, pl.ds(base + kk * L, L)]
        pred = chunk > rms[k]
        rms[k] = jnp.where(pred, chunk, rms[k])
        rcs[k] = jnp.where(pred, gvec, rcs[k])
      return tuple(rms) + tuple(rcs)

    carry = lax.fori_loop(b * GPB, (b + 1) * GPB, scan_step,
                          tuple(rms) + tuple(rcs))
    rms = list(carry[:U])
    rcs = list(carry[U:])

  for r in range(RPT):
    pairs = [(rms[r * UR + k], rcs[r * UR + k] * (UR * L) + (k * L + iota))
             for k in range(UR)]
    while len(pairs) > 1:
      nxt = []
      for a in range(0, len(pairs), 2):
        (m1, i1), (m2, i2) = pairs[a], pairs[a + 1]
        pred = (m2 > m1) | ((m2 == m1) & (i2 < i1))
        nxt.append((jnp.where(pred, m2, m1), jnp.where(pred, i2, i1)))
      pairs = nxt
    rm, ri = pairs[0]

    for sh in (8, 4, 2, 1):
      perm = (iota + sh) & (L - 1)
      other_m = _lane_shuffle(rm, perm)
      other_i = _lane_shuffle(ri, perm)
      pred = (other_m > rm) | ((other_m == rm) & (other_i < ri))
      rm = jnp.where(pred, other_m, rm)
      ri = jnp.where(pred, other_i, ri)

    val_v[r, pl.ds(0, L)] = rm
    idx_v[r, pl.ds(0, L)] = ri

  pltpu.sync_copy(val_v, vals_hbm.at[pl.ds(row0, RPT)])
  pltpu.sync_copy(idx_v, idxs_hbm.at[pl.ds(row0, RPT)])


def _tc_body(sc_vals_ref, sc_idxs_ref, x_ref, vals_ref, idxs_ref):
  j = pl.program_id(0)

  @pl.when(j == 0)
  def _():
    vals_ref[...] = sc_vals_ref[:, :1]
    idxs_ref[...] = sc_idxs_ref[:, :1]

  chunk = x_ref[...]
  lm = jnp.max(chunk, axis=1, keepdims=True)
  ii = lax.broadcasted_iota(jnp.int32, chunk.shape, 1)
  la = jnp.min(jnp.where(chunk == lm, ii, COLS), axis=1,
               keepdims=True) + (SC_COLS + j * TC_BLK)

  pred = lm > vals_ref[...]
  idxs_ref[...] = jnp.where(pred, la, idxs_ref[...])
  vals_ref[...] = jnp.where(pred, lm, vals_ref[...])


@jax.jit
def _topk1(x):
  mesh = plsc.VectorSubcoreMesh(core_axis_name="c", subcore_axis_name="s")
  sc_vals, sc_idxs = pl.kernel(
      _sc_body,
      out_type=(
          jax.ShapeDtypeStruct((ROWS, L), jnp.float32),
          jax.ShapeDtypeStruct((ROWS, L), jnp.int32),
      ),
      mesh=mesh,
      scratch_types=[
          pltpu.VMEM((RPT, SC_COLS), jnp.float32),
          pltpu.VMEM((RPT, L), jnp.float32),
          pltpu.VMEM((RPT, L), jnp.int32),
      ] + [pltpu.SemaphoreType.DMA] * (RPT * NB),
  )(x)

  vals, idxs = pl.pallas_call(
      _tc_body,
      grid=(TC_NCB,),
      in_specs=[pl.BlockSpec((ROWS, L), lambda j: (0, 0)),
                pl.BlockSpec((ROWS, L), lambda j: (0, 0)),
                pl.BlockSpec((ROWS, TC_BLK),
                             lambda j: (0, j + SC_COLS // TC_BLK))],
      out_specs=[pl.BlockSpec((ROWS, 1), lambda j: (0, 0)),
                 pl.BlockSpec((ROWS, 1), lambda j: (0, 0))],
      out_shape=(
          jax.ShapeDtypeStruct((ROWS, 1), jnp.float32),
          jax.ShapeDtypeStruct((ROWS, 1), jnp.int32),
      ),
      compiler_params=pltpu.CompilerParams(
          dimension_semantics=("arbitrary",)),
  )(sc_vals, sc_idxs, x)

  return vals, idxs


def kernel(x):
  return _topk1(x)


# speedup vs baseline: 1.1712x; 1.1712x over previous
import jax
import jax.numpy as jnp
from jax import lax
from jax.experimental import pallas as pl
from jax.experimental.pallas import tpu as pltpu
from jax.experimental.pallas import tpu_sc as plsc

NC = 2
NS = 16
L = 16
NW = NC * NS
ROWS = 64
COLS = 32768

SC_COLS = 8192
RPT = ROWS // NW

UR = 8
U = UR * RPT
NB = 2
BLK = SC_COLS // NB
GPB = BLK // (UR * L)

TC_BLK = 4096
TC_NCB = (COLS - SC_COLS) // TC_BLK


def _lane_shuffle(v, perm):
  dnums = lax.GatherDimensionNumbers(
      offset_dims=(), collapsed_slice_dims=(0,), start_index_map=(0,))
  return lax.gather(v, perm[:, None], dnums, slice_sizes=(1,),
                    mode=lax.GatherScatterMode.PROMISE_IN_BOUNDS)


def _sc_body(x_hbm, vals_hbm, idxs_hbm, xv, val_v, idx_v, *sems):
  cid = lax.axis_index("c")
  sid = lax.axis_index("s")
  row0 = (cid * NS + sid) * RPT

  descs = []
  for r in range(RPT):
    for b in range(NB):
      d = pltpu.async_copy(
          x_hbm.at[pl.ds(row0 + r, 1), pl.ds(b * BLK, BLK)],
          xv.at[pl.ds(r, 1), pl.ds(b * BLK, BLK)],
          sems[r * NB + b])
      descs.append(d)

  iota = lax.iota(jnp.int32, L)

  rms = [jnp.full((L,), -jnp.inf, jnp.float32) for _ in range(U)]
  rcs = [jnp.zeros((L,), jnp.int32) for _ in range(U)]

  for b in range(NB):
    for r in range(RPT):
      descs[r * NB + b].wait()

    def scan_step(g, carry):
      rms = list(carry[:U])
      rcs = list(carry[U:])
      gvec = jnp.zeros((L,), jnp.int32) + g
      base = g * (UR * L)
      for k in range(U):
        r, kk = divmod(k, UR)
        chunk = xv[r, pl.ds(base + kk * L, L)]
        pred = chunk > rms[k]
        rms[k] = jnp.where(pred, chunk, rms[k])
        rcs[k] = jnp.where(pred, gvec, rcs[k])
      return tuple(rms) + tuple(rcs)

    carry = lax.fori_loop(b * GPB, (b + 1) * GPB, scan_step,
                          tuple(rms) + tuple(rcs))
    rms = list(carry[:U])
    rcs = list(carry[U:])

  for r in range(RPT):
    pairs = [(rms[r * UR + k], rcs[r * UR + k] * (UR * L) + (k * L + iota))
             for k in range(UR)]
    while len(pairs) > 1:
      nxt = []
      for a in range(0, len(pairs), 2):
        (m1, i1), (m2, i2) = pairs[a], pairs[a + 1]
        pred = (m2 > m1) | ((m2 == m1) & (i2 < i1))
        nxt.append((jnp.where(pred, m2, m1), jnp.where(pred, i2, i1)))
      pairs = nxt
    rm, ri = pairs[0]

    for sh in (8, 4, 2, 1):
      perm = (iota + sh) & (L - 1)
      other_m = _lane_shuffle(rm, perm)
      other_i = _lane_shuffle(ri, perm)
      pred = (other_m > rm) | ((other_m == rm) & (other_i < ri))
      rm = jnp.where(pred, other_m, rm)
      ri = jnp.where(pred, other_i, ri)

    val_v[r, pl.ds(0, L)] = rm
    idx_v[r, pl.ds(0, L)] = ri

  pltpu.sync_copy(val_v, vals_hbm.at[pl.ds(row0, RPT)])
  pltpu.sync_copy(idx_v, idxs_hbm.at[pl.ds(row0, RPT)])


def _tc_body(x_ref, vals_ref, idxs_ref):
  j = pl.program_id(0)
  chunk = x_ref[...]
  lm = jnp.max(chunk, axis=1, keepdims=True)
  ii = lax.broadcasted_iota(jnp.int32, chunk.shape, 1)
  la = jnp.min(jnp.where(chunk == lm, ii, COLS), axis=1,
               keepdims=True) + (SC_COLS + j * TC_BLK)

  @pl.when(j == 0)
  def _():
    vals_ref[...] = lm
    idxs_ref[...] = la

  @pl.when(j != 0)
  def _():
    pred = lm > vals_ref[...]
    idxs_ref[...] = jnp.where(pred, la, idxs_ref[...])
    vals_ref[...] = jnp.where(pred, lm, vals_ref[...])


def _merge_body(sv_ref, si_ref, tv_ref, ti_ref, vals_ref, idxs_ref):
  sv = sv_ref[:, :1]
  si = si_ref[:, :1]
  tv = tv_ref[...]
  ti = ti_ref[...]
  pred = tv > sv
  vals_ref[...] = jnp.where(pred, tv, sv)
  idxs_ref[...] = jnp.where(pred, ti, si)


@jax.jit
def _topk1(x):
  mesh = plsc.VectorSubcoreMesh(core_axis_name="c", subcore_axis_name="s")
  sc_vals, sc_idxs = pl.kernel(
      _sc_body,
      out_type=(
          jax.ShapeDtypeStruct((ROWS, L), jnp.float32),
          jax.ShapeDtypeStruct((ROWS, L), jnp.int32),
      ),
      mesh=mesh,
      scratch_types=[
          pltpu.VMEM((RPT, SC_COLS), jnp.float32),
          pltpu.VMEM((RPT, L), jnp.float32),
          pltpu.VMEM((RPT, L), jnp.int32),
      ] + [pltpu.SemaphoreType.DMA] * (RPT * NB),
  )(x)

  tc_vals, tc_idxs = pl.pallas_call(
      _tc_body,
      grid=(TC_NCB,),
      in_specs=[pl.BlockSpec((ROWS, TC_BLK),
                             lambda j: (0, j + SC_COLS // TC_BLK))],
      out_specs=[pl.BlockSpec((ROWS, 1), lambda j: (0, 0)),
                 pl.BlockSpec((ROWS, 1), lambda j: (0, 0))],
      out_shape=(
          jax.ShapeDtypeStruct((ROWS, 1), jnp.float32),
          jax.ShapeDtypeStruct((ROWS, 1), jnp.int32),
      ),
      compiler_params=pltpu.CompilerParams(
          dimension_semantics=("arbitrary",)),
  )(x)

  vals, idxs = pl.pallas_call(
      _merge_body,
      out_shape=(
          jax.ShapeDtypeStruct((ROWS, 1), jnp.float32),
          jax.ShapeDtypeStruct((ROWS, 1), jnp.int32),
      ),
  )(sc_vals, sc_idxs, tc_vals, tc_idxs)

  return vals, idxs


def kernel(x):
  return _topk1(x)


# speedup vs baseline: 1.2368x; 1.0559x over previous
import jax
import jax.numpy as jnp
from jax import lax
from jax.experimental import pallas as pl
from jax.experimental.pallas import tpu as pltpu
from jax.experimental.pallas import tpu_sc as plsc

NC = 2
NS = 16
L = 16
NW = NC * NS
ROWS = 64
COLS = 32768

SC_COLS = 8192
RPT = ROWS // NW

UR = 8
U = UR * RPT
NB = 2
BLK = SC_COLS // NB
GPB = BLK // (UR * L)

TC_BLK = 8192
TC_NCB = (COLS - SC_COLS) // TC_BLK


def _lane_shuffle(v, perm):
  dnums = lax.GatherDimensionNumbers(
      offset_dims=(), collapsed_slice_dims=(0,), start_index_map=(0,))
  return lax.gather(v, perm[:, None], dnums, slice_sizes=(1,),
                    mode=lax.GatherScatterMode.PROMISE_IN_BOUNDS)


def _sc_body(x_hbm, vals_hbm, idxs_hbm, xv, val_v, idx_v, *sems):
  cid = lax.axis_index("c")
  sid = lax.axis_index("s")
  row0 = (cid * NS + sid) * RPT

  descs = []
  for r in range(RPT):
    for b in range(NB):
      d = pltpu.async_copy(
          x_hbm.at[pl.ds(row0 + r, 1), pl.ds(b * BLK, BLK)],
          xv.at[pl.ds(r, 1), pl.ds(b * BLK, BLK)],
          sems[r * NB + b])
      descs.append(d)

  iota = lax.iota(jnp.int32, L)

  rms = [jnp.full((L,), -jnp.inf, jnp.float32) for _ in range(U)]
  rcs = [jnp.zeros((L,), jnp.int32) for _ in range(U)]

  for b in range(NB):
    for r in range(RPT):
      descs[r * NB + b].wait()

    def scan_step(g, carry):
      rms = list(carry[:U])
      rcs = list(carry[U:])
      gvec = jnp.zeros((L,), jnp.int32) + g
      base = g * (UR * L)
      for k in range(U):
        r, kk = divmod(k, UR)
        chunk = xv[r, pl.ds(base + kk * L, L)]
        pred = chunk > rms[k]
        rms[k] = jnp.where(pred, chunk, rms[k])
        rcs[k] = jnp.where(pred, gvec, rcs[k])
      return tuple(rms) + tuple(rcs)

    carry = lax.fori_loop(b * GPB, (b + 1) * GPB, scan_step,
                          tuple(rms) + tuple(rcs))
    rms = list(carry[:U])
    rcs = list(carry[U:])

  for r in range(RPT):
    pairs = [(rms[r * UR + k], rcs[r * UR + k] * (UR * L) + (k * L + iota))
             for k in range(UR)]
    while len(pairs) > 1:
      nxt = []
      for a in range(0, len(pairs), 2):
        (m1, i1), (m2, i2) = pairs[a], pairs[a + 1]
        pred = (m2 > m1) | ((m2 == m1) & (i2 < i1))
        nxt.append((jnp.where(pred, m2, m1), jnp.where(pred, i2, i1)))
      pairs = nxt
    rm, ri = pairs[0]

    for sh in (8, 4, 2, 1):
      perm = (iota + sh) & (L - 1)
      other_m = _lane_shuffle(rm, perm)
      other_i = _lane_shuffle(ri, perm)
      pred = (other_m > rm) | ((other_m == rm) & (other_i < ri))
      rm = jnp.where(pred, other_m, rm)
      ri = jnp.where(pred, other_i, ri)

    val_v[r, pl.ds(0, L)] = rm
    idx_v[r, pl.ds(0, L)] = ri

  pltpu.sync_copy(val_v, vals_hbm.at[pl.ds(row0, RPT)])
  pltpu.sync_copy(idx_v, idxs_hbm.at[pl.ds(row0, RPT)])


def _tc_body(x_ref, vals_ref, idxs_ref):
  j = pl.program_id(0)
  chunk = x_ref[...]
  lm = jnp.max(chunk, axis=1, keepdims=True)
  ii = lax.broadcasted_iota(jnp.int32, chunk.shape, 1)
  la = jnp.min(jnp.where(chunk == lm, ii, COLS), axis=1,
               keepdims=True) + (SC_COLS + j * TC_BLK)

  @pl.when(j == 0)
  def _():
    vals_ref[...] = lm
    idxs_ref[...] = la

  @pl.when(j != 0)
  def _():
    pred = lm > vals_ref[...]
    idxs_ref[...] = jnp.where(pred, la, idxs_ref[...])
    vals_ref[...] = jnp.where(pred, lm, vals_ref[...])


@jax.jit
def _topk1(x):
  mesh = plsc.VectorSubcoreMesh(core_axis_name="c", subcore_axis_name="s")
  sc_vals, sc_idxs = pl.kernel(
      _sc_body,
      out_type=(
          jax.ShapeDtypeStruct((ROWS, L), jnp.float32),
          jax.ShapeDtypeStruct((ROWS, L), jnp.int32),
      ),
      mesh=mesh,
      scratch_types=[
          pltpu.VMEM((RPT, SC_COLS), jnp.float32),
          pltpu.VMEM((RPT, L), jnp.float32),
          pltpu.VMEM((RPT, L), jnp.int32),
      ] + [pltpu.SemaphoreType.DMA] * (RPT * NB),
  )(x)

  tc_vals, tc_idxs = pl.pallas_call(
      _tc_body,
      grid=(TC_NCB,),
      in_specs=[pl.BlockSpec((ROWS, TC_BLK),
                             lambda j: (0, j + SC_COLS // TC_BLK))],
      out_specs=[pl.BlockSpec((ROWS, 1), lambda j: (0, 0)),
                 pl.BlockSpec((ROWS, 1), lambda j: (0, 0))],
      out_shape=(
          jax.ShapeDtypeStruct((ROWS, 1), jnp.float32),
          jax.ShapeDtypeStruct((ROWS, 1), jnp.int32),
      ),
      compiler_params=pltpu.CompilerParams(
          dimension_semantics=("arbitrary",)),
  )(x)

  sv = sc_vals[:, :1]
  si = sc_idxs[:, :1]
  pred = tc_vals > sv
  vals = jnp.where(pred, tc_vals, sv)
  idxs = jnp.where(pred, tc_idxs, si)
  return vals, idxs


def kernel(x):
  return _topk1(x)
